# baseline (speedup 1.0000x reference)
import functools

import jax
import jax.numpy as jnp
from jax import lax
from jax.experimental import pallas as pl
from jax.experimental.pallas import tpu as pltpu
from jax.experimental.pallas import tpu_sc as plsc

T, D, I, E, TOP_K = 2048, 1024, 1024, 8, 2
SWIGLU_LIMIT = 7.0
SWIGLU_ALPHA = 1.702

NPAIR = T * TOP_K
BM = 512
NB = NPAIR // BM + E
NP = NB * BM

NC, NS = 2, 16
NW = NC * NS
TPW = T // NW

@functools.cache
def _sc_mesh():
    return plsc.VectorSubcoreMesh(core_axis_name="c", subcore_axis_name="s")


_RBM = 512


_PPB = 2 * _RBM
_TRI = None


def _tri_const():
    global _TRI
    if _TRI is None:
        import numpy as _np
        _TRI = jnp.asarray(
            _np.tril(_np.ones((_PPB, _PPB), _np.float32), -1), jnp.bfloat16)
    return _TRI


def _router_body(x_ref, wgt_ref, bg_ref, tri_ref,
                 i1_ref, i2_ref, w0_ref, w1_ref, re_ref, ro_ref, cnt_ref,
                 acc_ref):
    m = pl.program_id(0)

    @pl.when(m == 0)
    def _():
        acc_ref[...] = jnp.zeros_like(acc_ref)

    logits = (
        jnp.dot(x_ref[...], wgt_ref[...], preferred_element_type=jnp.float32)
        + bg_ref[...]
    )
    lane = lax.broadcasted_iota(jnp.int32, (_RBM, E), 1)
    v1 = jnp.max(logits, axis=1, keepdims=True)
    i1 = jnp.min(jnp.where(logits == v1, lane, E), axis=1, keepdims=True)
    masked = jnp.where(lane == i1, -jnp.inf, logits)
    v2 = jnp.max(masked, axis=1, keepdims=True)
    i2 = jnp.min(jnp.where(masked == v2, lane, E), axis=1, keepdims=True)
    w0 = 1.0 / (1.0 + jnp.exp(v2 - v1))
    i1_ref[...] = i1
    i2_ref[...] = i2
    w0_ref[...] = w0
    w1_ref[...] = 1.0 - w0

    oh = jnp.concatenate(
        [(lane == i1).astype(jnp.float32), (lane == i2).astype(jnp.float32)],
        axis=0)
    r_local = jnp.dot(tri_ref[...], oh.astype(jnp.bfloat16),
                      preferred_element_type=jnp.float32)
    acc = acc_ref[...]
    ranksel = jnp.sum((r_local + acc) * oh, axis=1, keepdims=True)
    re_ref[...] = ranksel[:_RBM].astype(jnp.int32)
    ro_ref[...] = ranksel[_RBM:].astype(jnp.int32)
    new_acc = acc + jnp.sum(oh, axis=0, keepdims=True)
    acc_ref[...] = new_acc
    cnt_ref[...] = new_acc.astype(jnp.int32)


def _router(x, Wg, bg):
    return pl.pallas_call(
        _router_body,
        grid=(T // _RBM,),
        in_specs=[
            pl.BlockSpec((_RBM, D), lambda m: (m, 0)),
            pl.BlockSpec((D, E), lambda m: (0, 0)),
            pl.BlockSpec((1, E), lambda m: (0, 0)),
            pl.BlockSpec((_PPB, _PPB), lambda m: (0, 0)),
        ],
        out_specs=[
            pl.BlockSpec((_RBM, 1), lambda m: (m, 0)),
            pl.BlockSpec((_RBM, 1), lambda m: (m, 0)),
            pl.BlockSpec((_RBM, 1), lambda m: (m, 0)),
            pl.BlockSpec((_RBM, 1), lambda m: (m, 0)),
            pl.BlockSpec((_RBM, 1), lambda m: (m, 0)),
            pl.BlockSpec((_RBM, 1), lambda m: (m, 0)),
            pl.BlockSpec((1, E), lambda m: (0, 0)),
        ],
        out_shape=[
            jax.ShapeDtypeStruct((T, 1), jnp.int32),
            jax.ShapeDtypeStruct((T, 1), jnp.int32),
            jax.ShapeDtypeStruct((T, 1), jnp.float32),
            jax.ShapeDtypeStruct((T, 1), jnp.float32),
            jax.ShapeDtypeStruct((T, 1), jnp.int32),
            jax.ShapeDtypeStruct((T, 1), jnp.int32),
            jax.ShapeDtypeStruct((1, E), jnp.int32),
        ],
        scratch_shapes=[pltpu.VMEM((1, E), jnp.float32)],
    )(x, Wg.T, bg.reshape(1, E), _tri_const())


def _dispatch_body(x_ref, de_ref, do_ref, xs_ref,
                   rows_v, de_v, do_v, sem0, sem1):
    wid = lax.axis_index("s") * NC + lax.axis_index("c")
    base = wid * TPW
    pltpu.sync_copy(de_ref.at[wid], de_v)
    pltpu.sync_copy(do_ref.at[wid], do_v)
    pltpu.sync_copy(x_ref.at[pl.ds(base, TPW)], rows_v)
    c0 = pltpu.async_copy(rows_v, xs_ref.at[de_v], sem0)
    c1 = pltpu.async_copy(rows_v, xs_ref.at[do_v], sem1)
    c0.wait()
    c1.wait()


@functools.cache
def _dispatch_kernel():
    return pl.kernel(
        _dispatch_body, mesh=_sc_mesh(),
        out_type=jax.ShapeDtypeStruct((NP, D), jnp.float32),
        scratch_types=[
            pltpu.VMEM((TPW, D), jnp.float32),
            pltpu.VMEM((TPW,), jnp.int32),
            pltpu.VMEM((TPW,), jnp.int32),
            pltpu.SemaphoreType.DMA,
            pltpu.SemaphoreType.DMA,
        ],
    )


def _dispatch(xb, de, do):
    return _dispatch_kernel()(xb, de, do)


CCH = 32


def _cgather_body(y_ref, p0_ref, p1_ref, g0_ref, g1_ref,
                  b0_v, b1_v, p0_v, p1_v, sem0, sem1):
    wid = lax.axis_index("s") * NC + lax.axis_index("c")
    base = wid * TPW
    for c in range(TPW // CCH):
        pltpu.sync_copy(p0_ref.at[wid, pl.ds(c * CCH, CCH)], p0_v)
        pltpu.sync_copy(p1_ref.at[wid, pl.ds(c * CCH, CCH)], p1_v)
        c0 = pltpu.async_copy(y_ref.at[p0_v], b0_v, sem0)
        c1 = pltpu.async_copy(y_ref.at[p1_v], b1_v, sem1)
        c0.wait()
        c1.wait()
        pltpu.sync_copy(b0_v, g0_ref.at[pl.ds(base + c * CCH, CCH)])
        pltpu.sync_copy(b1_v, g1_ref.at[pl.ds(base + c * CCH, CCH)])


@functools.cache
def _cgather_kernel():
    return pl.kernel(
        _cgather_body, mesh=_sc_mesh(),
        out_type=(jax.ShapeDtypeStruct((T, D), jnp.float32),
                  jax.ShapeDtypeStruct((T, D), jnp.float32)),
        scratch_types=[
            pltpu.VMEM((CCH, D), jnp.float32),
            pltpu.VMEM((CCH, D), jnp.float32),
            pltpu.VMEM((CCH,), jnp.int32),
            pltpu.VMEM((CCH,), jnp.int32),
            pltpu.SemaphoreType.DMA,
            pltpu.SemaphoreType.DMA,
        ],
    )


def _cgather(y, de, do):
    return _cgather_kernel()(y, de, do)


_CBM = 256


def _wsum_body(g0_ref, g1_ref, w0_ref, w1_ref, o_ref):
    o_ref[...] = w0_ref[...] * g0_ref[...] + w1_ref[...] * g1_ref[...]


def _wsum(g0, g1, w0, w1):
    return pl.pallas_call(
        _wsum_body,
        grid=(T // _CBM,),
        in_specs=[
            pl.BlockSpec((_CBM, D), lambda m: (m, 0)),
            pl.BlockSpec((_CBM, D), lambda m: (m, 0)),
            pl.BlockSpec((_CBM, 1), lambda m: (m, 0)),
            pl.BlockSpec((_CBM, 1), lambda m: (m, 0)),
        ],
        out_specs=pl.BlockSpec((_CBM, D), lambda m: (m, 0)),
        out_shape=jax.ShapeDtypeStruct((T, D), jnp.float32),
    )(g0, g1, w0, w1)


def _gmm_body(bexp_ref, nblk_ref, x_ref, w1_ref, b1_ref, w2_ref, b2_ref,
              o_ref):
    m = pl.program_id(0)

    @pl.when(m < nblk_ref[0])
    def _():
        h = lax.dot_general(
            x_ref[...].astype(jnp.bfloat16), w1_ref[0],
            (((1,), (0,)), ((), ())),
            preferred_element_type=jnp.float32,
        ) + b1_ref[0]
        gate = jnp.minimum(h[:, :I], SWIGLU_LIMIT)
        up = jnp.clip(h[:, I:], -SWIGLU_LIMIT, SWIGLU_LIMIT)
        act = gate * jax.nn.sigmoid(SWIGLU_ALPHA * gate) * (up + 1.0)
        y = lax.dot_general(
            act.astype(jnp.bfloat16), w2_ref[0],
            (((1,), (0,)), ((), ())),
            preferred_element_type=jnp.float32,
        ) + b2_ref[0]
        o_ref[...] = y


def _gmm(x_sorted, W1b, b1, W2b, b2, bexp, nblk):
    grid_spec = pltpu.PrefetchScalarGridSpec(
        num_scalar_prefetch=2,
        grid=(NB,),
        in_specs=[
            pl.BlockSpec((BM, D), lambda m, be, nb: (m, 0)),
            pl.BlockSpec((1, D, 2 * I), lambda m, be, nb: (be[m], 0, 0)),
            pl.BlockSpec((1, 1, 2 * I), lambda m, be, nb: (be[m], 0, 0)),
            pl.BlockSpec((1, I, D), lambda m, be, nb: (be[m], 0, 0)),
            pl.BlockSpec((1, 1, D), lambda m, be, nb: (be[m], 0, 0)),
        ],
        out_specs=pl.BlockSpec((BM, D), lambda m, be, nb: (m, 0)),
    )
    return pl.pallas_call(
        _gmm_body,
        grid_spec=grid_spec,
        out_shape=jax.ShapeDtypeStruct((NP, D), jnp.float32),
    )(bexp, nblk, x_sorted, W1b, b1, W2b, b2)


def kernel(hidden_states, Wg, bg, W1, b1, W2, b2):
    x = hidden_states
    i1, i2, w0, w1, re, ro, cnt = _router(x, Wg, bg)

    counts = cnt.reshape(E)
    blocks_per_e = (counts + BM - 1) // BM
    block_end = jnp.cumsum(blocks_per_e)
    row_start = (block_end - blocks_per_e) * BM
    nblk = block_end[-1:].astype(jnp.int32)
    m_idx = jnp.arange(NB, dtype=jnp.int32)
    bexp = jnp.minimum(
        jnp.sum((m_idx[:, None] >= block_end[None, :]).astype(jnp.int32), 1),
        E - 1,
    )

    eye = jnp.arange(E, dtype=jnp.int32)[None, :]
    rs1 = jnp.sum(jnp.where(i1 == eye, row_start[None, :], 0), axis=1)
    rs2 = jnp.sum(jnp.where(i2 == eye, row_start[None, :], 0), axis=1)
    de = (rs1 + re[:, 0]).reshape(NW, TPW)
    do = (rs2 + ro[:, 0]).reshape(NW, TPW)

    x_sorted = _dispatch(x, de, do)

    y = _gmm(x_sorted, W1.astype(jnp.bfloat16), b1.reshape(E, 1, 2 * I),
             W2.astype(jnp.bfloat16), b2.reshape(E, 1, D), bexp, nblk)

    g0, g1 = _cgather(y, de, do)
    return _wsum(g0, g1, w0, w1)


# speedup vs baseline: 1.0005x; 1.0005x over previous
import functools

import jax
import jax.numpy as jnp
from jax import lax
from jax.experimental import pallas as pl
from jax.experimental.pallas import tpu as pltpu
from jax.experimental.pallas import tpu_sc as plsc

T, D, I, E, TOP_K = 2048, 1024, 1024, 8, 2
SWIGLU_LIMIT = 7.0
SWIGLU_ALPHA = 1.702

NPAIR = T * TOP_K
BM = 512
NB = NPAIR // BM + E
NP = NB * BM

NC, NS = 2, 16
NW = NC * NS
TPW = T // NW

@functools.cache
def _sc_mesh():
    return plsc.VectorSubcoreMesh(core_axis_name="c", subcore_axis_name="s")


_RBM = 512


_PPB = 2 * _RBM
_TRI = None


def _tri_const():
    global _TRI
    if _TRI is None:
        import numpy as _np
        _TRI = jnp.asarray(
            _np.tril(_np.ones((_PPB, _PPB), _np.float32), -1), jnp.bfloat16)
    return _TRI


def _router_body(x_ref, wgt_ref, bg_ref, tri_ref,
                 i1_ref, i2_ref, w0_ref, w1_ref, re_ref, ro_ref, cnt_ref,
                 acc_ref):
    m = pl.program_id(0)

    @pl.when(m == 0)
    def _():
        acc_ref[...] = jnp.zeros_like(acc_ref)

    logits = (
        jnp.dot(x_ref[...], wgt_ref[...], preferred_element_type=jnp.float32)
        + bg_ref[...]
    )
    lane = lax.broadcasted_iota(jnp.int32, (_RBM, E), 1)
    v1 = jnp.max(logits, axis=1, keepdims=True)
    i1 = jnp.min(jnp.where(logits == v1, lane, E), axis=1, keepdims=True)
    masked = jnp.where(lane == i1, -jnp.inf, logits)
    v2 = jnp.max(masked, axis=1, keepdims=True)
    i2 = jnp.min(jnp.where(masked == v2, lane, E), axis=1, keepdims=True)
    w0 = 1.0 / (1.0 + jnp.exp(v2 - v1))
    i1_ref[...] = i1
    i2_ref[...] = i2
    w0_ref[...] = w0
    w1_ref[...] = 1.0 - w0

    oh = jnp.concatenate(
        [(lane == i1).astype(jnp.float32), (lane == i2).astype(jnp.float32)],
        axis=0)
    r_local = jnp.dot(tri_ref[...], oh.astype(jnp.bfloat16),
                      preferred_element_type=jnp.float32)
    acc = acc_ref[...]
    ranksel = jnp.sum((r_local + acc) * oh, axis=1, keepdims=True)
    re_ref[...] = ranksel[:_RBM].astype(jnp.int32)
    ro_ref[...] = ranksel[_RBM:].astype(jnp.int32)
    new_acc = acc + jnp.sum(oh, axis=0, keepdims=True)
    acc_ref[...] = new_acc
    cnt_ref[...] = new_acc.astype(jnp.int32)


def _router(x, Wg, bg):
    return pl.pallas_call(
        _router_body,
        grid=(T // _RBM,),
        in_specs=[
            pl.BlockSpec((_RBM, D), lambda m: (m, 0)),
            pl.BlockSpec((D, E), lambda m: (0, 0)),
            pl.BlockSpec((1, E), lambda m: (0, 0)),
            pl.BlockSpec((_PPB, _PPB), lambda m: (0, 0)),
        ],
        out_specs=[
            pl.BlockSpec((_RBM, 1), lambda m: (m, 0)),
            pl.BlockSpec((_RBM, 1), lambda m: (m, 0)),
            pl.BlockSpec((_RBM, 1), lambda m: (m, 0)),
            pl.BlockSpec((_RBM, 1), lambda m: (m, 0)),
            pl.BlockSpec((_RBM, 1), lambda m: (m, 0)),
            pl.BlockSpec((_RBM, 1), lambda m: (m, 0)),
            pl.BlockSpec((1, E), lambda m: (0, 0)),
        ],
        out_shape=[
            jax.ShapeDtypeStruct((T, 1), jnp.int32),
            jax.ShapeDtypeStruct((T, 1), jnp.int32),
            jax.ShapeDtypeStruct((T, 1), jnp.float32),
            jax.ShapeDtypeStruct((T, 1), jnp.float32),
            jax.ShapeDtypeStruct((T, 1), jnp.int32),
            jax.ShapeDtypeStruct((T, 1), jnp.int32),
            jax.ShapeDtypeStruct((1, E), jnp.int32),
        ],
        scratch_shapes=[pltpu.VMEM((1, E), jnp.float32)],
    )(x, Wg.T, bg.reshape(1, E), _tri_const())


def _dispatch_body(x_ref, de_ref, do_ref, xs_ref,
                   rows_v, de_v, do_v, sem0, sem1):
    wid = lax.axis_index("s") * NC + lax.axis_index("c")
    base = wid * TPW
    pltpu.sync_copy(de_ref.at[wid], de_v)
    pltpu.sync_copy(do_ref.at[wid], do_v)
    pltpu.sync_copy(x_ref.at[pl.ds(base, TPW)], rows_v)
    c0 = pltpu.async_copy(rows_v, xs_ref.at[de_v], sem0)
    c1 = pltpu.async_copy(rows_v, xs_ref.at[do_v], sem1)
    c0.wait()
    c1.wait()


@functools.cache
def _dispatch_kernel():
    return pl.kernel(
        _dispatch_body, mesh=_sc_mesh(),
        out_type=jax.ShapeDtypeStruct((NP, D), jnp.float32),
        scratch_types=[
            pltpu.VMEM((TPW, D), jnp.float32),
            pltpu.VMEM((TPW,), jnp.int32),
            pltpu.VMEM((TPW,), jnp.int32),
            pltpu.SemaphoreType.DMA,
            pltpu.SemaphoreType.DMA,
        ],
    )


def _dispatch(xb, de, do):
    return _dispatch_kernel()(xb, de, do)


CCH = 32


def _cgather_body(y_ref, p0_ref, p1_ref, g0_ref, g1_ref,
                  b0_v, b1_v, p0_v, p1_v, sem0, sem1):
    wid = lax.axis_index("s") * NC + lax.axis_index("c")
    base = wid * TPW
    for c in range(TPW // CCH):
        pltpu.sync_copy(p0_ref.at[wid, pl.ds(c * CCH, CCH)], p0_v)
        pltpu.sync_copy(p1_ref.at[wid, pl.ds(c * CCH, CCH)], p1_v)
        c0 = pltpu.async_copy(y_ref.at[p0_v], b0_v, sem0)
        c1 = pltpu.async_copy(y_ref.at[p1_v], b1_v, sem1)
        c0.wait()
        c1.wait()
        pltpu.sync_copy(b0_v, g0_ref.at[pl.ds(base + c * CCH, CCH)])
        pltpu.sync_copy(b1_v, g1_ref.at[pl.ds(base + c * CCH, CCH)])


@functools.cache
def _cgather_kernel():
    return pl.kernel(
        _cgather_body, mesh=_sc_mesh(),
        out_type=(jax.ShapeDtypeStruct((T, D), jnp.float32),
                  jax.ShapeDtypeStruct((T, D), jnp.float32)),
        scratch_types=[
            pltpu.VMEM((CCH, D), jnp.float32),
            pltpu.VMEM((CCH, D), jnp.float32),
            pltpu.VMEM((CCH,), jnp.int32),
            pltpu.VMEM((CCH,), jnp.int32),
            pltpu.SemaphoreType.DMA,
            pltpu.SemaphoreType.DMA,
        ],
    )


def _cgather(y, de, do):
    return _cgather_kernel()(y, de, do)


_CBM = 256


def _wsum_body(g0_ref, g1_ref, w0_ref, w1_ref, o_ref):
    o_ref[...] = w0_ref[...] * g0_ref[...] + w1_ref[...] * g1_ref[...]


def _wsum(g0, g1, w0, w1):
    return pl.pallas_call(
        _wsum_body,
        grid=(T // _CBM,),
        in_specs=[
            pl.BlockSpec((_CBM, D), lambda m: (m, 0)),
            pl.BlockSpec((_CBM, D), lambda m: (m, 0)),
            pl.BlockSpec((_CBM, 1), lambda m: (m, 0)),
            pl.BlockSpec((_CBM, 1), lambda m: (m, 0)),
        ],
        out_specs=pl.BlockSpec((_CBM, D), lambda m: (m, 0)),
        out_shape=jax.ShapeDtypeStruct((T, D), jnp.float32),
    )(g0, g1, w0, w1)


def _gmm_body(bexp_ref, nblk_ref, x_ref, w1_ref, b1_ref, w2_ref, b2_ref,
              o_ref):
    m = pl.program_id(0)

    @pl.when(m < nblk_ref[0])
    def _():
        xb = x_ref[...].astype(jnp.bfloat16)
        gate = lax.dot_general(
            xb, w1_ref[0][:, :I],
            (((1,), (0,)), ((), ())),
            preferred_element_type=jnp.float32,
        ) + b1_ref[0][:, :I]
        up = lax.dot_general(
            xb, w1_ref[0][:, I:],
            (((1,), (0,)), ((), ())),
            preferred_element_type=jnp.float32,
        ) + b1_ref[0][:, I:]
        gate = jnp.minimum(gate, SWIGLU_LIMIT)
        up = jnp.clip(up, -SWIGLU_LIMIT, SWIGLU_LIMIT)
        act = gate * jax.nn.sigmoid(SWIGLU_ALPHA * gate) * (up + 1.0)
        y = lax.dot_general(
            act.astype(jnp.bfloat16), w2_ref[0],
            (((1,), (0,)), ((), ())),
            preferred_element_type=jnp.float32,
        ) + b2_ref[0]
        o_ref[...] = y


def _gmm(x_sorted, W1b, b1, W2b, b2, bexp, nblk):
    grid_spec = pltpu.PrefetchScalarGridSpec(
        num_scalar_prefetch=2,
        grid=(NB,),
        in_specs=[
            pl.BlockSpec((BM, D), lambda m, be, nb: (m, 0)),
            pl.BlockSpec((1, D, 2 * I), lambda m, be, nb: (be[m], 0, 0)),
            pl.BlockSpec((1, 1, 2 * I), lambda m, be, nb: (be[m], 0, 0)),
            pl.BlockSpec((1, I, D), lambda m, be, nb: (be[m], 0, 0)),
            pl.BlockSpec((1, 1, D), lambda m, be, nb: (be[m], 0, 0)),
        ],
        out_specs=pl.BlockSpec((BM, D), lambda m, be, nb: (m, 0)),
    )
    return pl.pallas_call(
        _gmm_body,
        grid_spec=grid_spec,
        out_shape=jax.ShapeDtypeStruct((NP, D), jnp.float32),
    )(bexp, nblk, x_sorted, W1b, b1, W2b, b2)


def kernel(hidden_states, Wg, bg, W1, b1, W2, b2):
    x = hidden_states
    i1, i2, w0, w1, re, ro, cnt = _router(x, Wg, bg)

    counts = cnt.reshape(E)
    blocks_per_e = (counts + BM - 1) // BM
    block_end = jnp.cumsum(blocks_per_e)
    row_start = (block_end - blocks_per_e) * BM
    nblk = block_end[-1:].astype(jnp.int32)
    m_idx = jnp.arange(NB, dtype=jnp.int32)
    bexp = jnp.minimum(
        jnp.sum((m_idx[:, None] >= block_end[None, :]).astype(jnp.int32), 1),
        E - 1,
    )

    eye = jnp.arange(E, dtype=jnp.int32)[None, :]
    rs1 = jnp.sum(jnp.where(i1 == eye, row_start[None, :], 0), axis=1)
    rs2 = jnp.sum(jnp.where(i2 == eye, row_start[None, :], 0), axis=1)
    de = (rs1 + re[:, 0]).reshape(NW, TPW)
    do = (rs2 + ro[:, 0]).reshape(NW, TPW)

    x_sorted = _dispatch(x, de, do)

    y = _gmm(x_sorted, W1.astype(jnp.bfloat16), b1.reshape(E, 1, 2 * I),
             W2.astype(jnp.bfloat16), b2.reshape(E, 1, D), bexp, nblk)

    g0, g1 = _cgather(y, de, do)
    return _wsum(g0, g1, w0, w1)


# speedup vs baseline: 1.0213x; 1.0208x over previous
import functools

import jax
import jax.numpy as jnp
from jax import lax
from jax.experimental import pallas as pl
from jax.experimental.pallas import tpu as pltpu
from jax.experimental.pallas import tpu_sc as plsc

T, D, I, E, TOP_K = 2048, 1024, 1024, 8, 2
SWIGLU_LIMIT = 7.0
SWIGLU_ALPHA = 1.702

NPAIR = T * TOP_K
BM = 512
NB = NPAIR // BM + E
NP = NB * BM

NC, NS = 2, 16
NW = NC * NS
TPW = T // NW

@functools.cache
def _sc_mesh():
    return plsc.VectorSubcoreMesh(core_axis_name="c", subcore_axis_name="s")


_RBM = 512


_PPB = 2 * _RBM
_TRI = None


def _tri_const():
    global _TRI
    if _TRI is None:
        import numpy as _np
        _TRI = jnp.asarray(
            _np.tril(_np.ones((_PPB, _PPB), _np.float32), -1), jnp.bfloat16)
    return _TRI


def _router_body(x_ref, wgt_ref, bg_ref, tri_ref,
                 i1_ref, i2_ref, w0_ref, w1_ref, re_ref, ro_ref, cnt_ref,
                 acc_ref):
    m = pl.program_id(0)

    @pl.when(m == 0)
    def _():
        acc_ref[...] = jnp.zeros_like(acc_ref)

    logits = (
        jnp.dot(x_ref[...], wgt_ref[...], preferred_element_type=jnp.float32)
        + bg_ref[...]
    )
    lane = lax.broadcasted_iota(jnp.int32, (_RBM, E), 1)
    v1 = jnp.max(logits, axis=1, keepdims=True)
    i1 = jnp.min(jnp.where(logits == v1, lane, E), axis=1, keepdims=True)
    masked = jnp.where(lane == i1, -jnp.inf, logits)
    v2 = jnp.max(masked, axis=1, keepdims=True)
    i2 = jnp.min(jnp.where(masked == v2, lane, E), axis=1, keepdims=True)
    w0 = 1.0 / (1.0 + jnp.exp(v2 - v1))
    i1_ref[...] = i1
    i2_ref[...] = i2
    ones16 = jnp.ones((1, 16), jnp.float32)
    w0_ref[...] = w0 * ones16
    w1_ref[...] = (1.0 - w0) * ones16

    oh = jnp.concatenate(
        [(lane == i1).astype(jnp.float32), (lane == i2).astype(jnp.float32)],
        axis=0)
    r_local = jnp.dot(tri_ref[...], oh.astype(jnp.bfloat16),
                      preferred_element_type=jnp.float32)
    acc = acc_ref[...]
    ranksel = jnp.sum((r_local + acc) * oh, axis=1, keepdims=True)
    re_ref[...] = ranksel[:_RBM].astype(jnp.int32)
    ro_ref[...] = ranksel[_RBM:].astype(jnp.int32)
    new_acc = acc + jnp.sum(oh, axis=0, keepdims=True)
    acc_ref[...] = new_acc
    cnt_ref[...] = new_acc.astype(jnp.int32)


def _router(x, Wg, bg):
    return pl.pallas_call(
        _router_body,
        grid=(T // _RBM,),
        in_specs=[
            pl.BlockSpec((_RBM, D), lambda m: (m, 0)),
            pl.BlockSpec((D, E), lambda m: (0, 0)),
            pl.BlockSpec((1, E), lambda m: (0, 0)),
            pl.BlockSpec((_PPB, _PPB), lambda m: (0, 0)),
        ],
        out_specs=[
            pl.BlockSpec((_RBM, 1), lambda m: (m, 0)),
            pl.BlockSpec((_RBM, 1), lambda m: (m, 0)),
            pl.BlockSpec((_RBM, 16), lambda m: (m, 0)),
            pl.BlockSpec((_RBM, 16), lambda m: (m, 0)),
            pl.BlockSpec((_RBM, 1), lambda m: (m, 0)),
            pl.BlockSpec((_RBM, 1), lambda m: (m, 0)),
            pl.BlockSpec((1, E), lambda m: (0, 0)),
        ],
        out_shape=[
            jax.ShapeDtypeStruct((T, 1), jnp.int32),
            jax.ShapeDtypeStruct((T, 1), jnp.int32),
            jax.ShapeDtypeStruct((T, 16), jnp.float32),
            jax.ShapeDtypeStruct((T, 16), jnp.float32),
            jax.ShapeDtypeStruct((T, 1), jnp.int32),
            jax.ShapeDtypeStruct((T, 1), jnp.int32),
            jax.ShapeDtypeStruct((1, E), jnp.int32),
        ],
        scratch_shapes=[pltpu.VMEM((1, E), jnp.float32)],
    )(x, Wg.T, bg.reshape(1, E), _tri_const())


def _dispatch_body(x_ref, de_ref, do_ref, xs_ref,
                   rows_v, de_v, do_v, sem0, sem1):
    wid = lax.axis_index("s") * NC + lax.axis_index("c")
    base = wid * TPW
    pltpu.sync_copy(de_ref.at[wid], de_v)
    pltpu.sync_copy(do_ref.at[wid], do_v)
    pltpu.sync_copy(x_ref.at[pl.ds(base, TPW)], rows_v)
    c0 = pltpu.async_copy(rows_v, xs_ref.at[de_v], sem0)
    c1 = pltpu.async_copy(rows_v, xs_ref.at[do_v], sem1)
    c0.wait()
    c1.wait()


@functools.cache
def _dispatch_kernel():
    return pl.kernel(
        _dispatch_body, mesh=_sc_mesh(),
        out_type=jax.ShapeDtypeStruct((NP, D), jnp.float32),
        scratch_types=[
            pltpu.VMEM((TPW, D), jnp.float32),
            pltpu.VMEM((TPW,), jnp.int32),
            pltpu.VMEM((TPW,), jnp.int32),
            pltpu.SemaphoreType.DMA,
            pltpu.SemaphoreType.DMA,
        ],
    )


def _dispatch(xb, de, do):
    return _dispatch_kernel()(xb, de, do)


CCH = 32


def _cgather_body(y_ref, p0_ref, p1_ref, w0_ref, w1_ref, o_ref,
                  b0_v, b1_v, p0_v, p1_v, w0_v, w1_v, sem0, sem1):
    wid = lax.axis_index("s") * NC + lax.axis_index("c")
    base = wid * TPW
    pltpu.sync_copy(w0_ref.at[wid], w0_v)
    pltpu.sync_copy(w1_ref.at[wid], w1_v)
    for c in range(TPW // CCH):
        pltpu.sync_copy(p0_ref.at[wid, pl.ds(c * CCH, CCH)], p0_v)
        pltpu.sync_copy(p1_ref.at[wid, pl.ds(c * CCH, CCH)], p1_v)
        c0 = pltpu.async_copy(y_ref.at[p0_v], b0_v, sem0)
        c1 = pltpu.async_copy(y_ref.at[p1_v], b1_v, sem1)
        c0.wait()
        c1.wait()

        def _wrow(r, carry):
            tok = c * CCH + r
            wv0 = w0_v[tok, pl.ds(0, 16)]
            wv1 = w1_v[tok, pl.ds(0, 16)]
            for j in range(D // 16):
                sl = pl.ds(j * 16, 16)
                b0_v[r, sl] = b0_v[r, sl] * wv0 + b1_v[r, sl] * wv1
            return carry

        lax.fori_loop(0, CCH, _wrow, 0)
        pltpu.sync_copy(b0_v, o_ref.at[pl.ds(base + c * CCH, CCH)])


@functools.cache
def _cgather_kernel():
    return pl.kernel(
        _cgather_body, mesh=_sc_mesh(),
        out_type=jax.ShapeDtypeStruct((T, D), jnp.float32),
        scratch_types=[
            pltpu.VMEM((CCH, D), jnp.float32),
            pltpu.VMEM((CCH, D), jnp.float32),
            pltpu.VMEM((CCH,), jnp.int32),
            pltpu.VMEM((CCH,), jnp.int32),
            pltpu.VMEM((TPW, 16), jnp.float32),
            pltpu.VMEM((TPW, 16), jnp.float32),
            pltpu.SemaphoreType.DMA,
            pltpu.SemaphoreType.DMA,
        ],
    )


def _cgather(y, de, do, w0, w1):
    return _cgather_kernel()(y, de, do, w0, w1)


_CBM = 256


def _wsum_body(g0_ref, g1_ref, w0_ref, w1_ref, o_ref):
    o_ref[...] = w0_ref[...] * g0_ref[...] + w1_ref[...] * g1_ref[...]


def _wsum(g0, g1, w0, w1):
    return pl.pallas_call(
        _wsum_body,
        grid=(T // _CBM,),
        in_specs=[
            pl.BlockSpec((_CBM, D), lambda m: (m, 0)),
            pl.BlockSpec((_CBM, D), lambda m: (m, 0)),
            pl.BlockSpec((_CBM, 1), lambda m: (m, 0)),
            pl.BlockSpec((_CBM, 1), lambda m: (m, 0)),
        ],
        out_specs=pl.BlockSpec((_CBM, D), lambda m: (m, 0)),
        out_shape=jax.ShapeDtypeStruct((T, D), jnp.float32),
    )(g0, g1, w0, w1)


def _gmm_body(bexp_ref, nblk_ref, x_ref, w1_ref, b1_ref, w2_ref, b2_ref,
              o_ref):
    m = pl.program_id(0)

    @pl.when(m < nblk_ref[0])
    def _():
        xb = x_ref[...].astype(jnp.bfloat16)
        gate = lax.dot_general(
            xb, w1_ref[0][:, :I],
            (((1,), (0,)), ((), ())),
            preferred_element_type=jnp.float32,
        ) + b1_ref[0][:, :I]
        up = lax.dot_general(
            xb, w1_ref[0][:, I:],
            (((1,), (0,)), ((), ())),
            preferred_element_type=jnp.float32,
        ) + b1_ref[0][:, I:]
        gate = jnp.minimum(gate, SWIGLU_LIMIT)
        up = jnp.clip(up, -SWIGLU_LIMIT, SWIGLU_LIMIT)
        act = gate * jax.nn.sigmoid(SWIGLU_ALPHA * gate) * (up + 1.0)
        y = lax.dot_general(
            act.astype(jnp.bfloat16), w2_ref[0],
            (((1,), (0,)), ((), ())),
            preferred_element_type=jnp.float32,
        ) + b2_ref[0]
        o_ref[...] = y


def _gmm(x_sorted, W1b, b1, W2b, b2, bexp, nblk):
    grid_spec = pltpu.PrefetchScalarGridSpec(
        num_scalar_prefetch=2,
        grid=(NB,),
        in_specs=[
            pl.BlockSpec((BM, D), lambda m, be, nb: (m, 0)),
            pl.BlockSpec((1, D, 2 * I), lambda m, be, nb: (be[m], 0, 0)),
            pl.BlockSpec((1, 1, 2 * I), lambda m, be, nb: (be[m], 0, 0)),
            pl.BlockSpec((1, I, D), lambda m, be, nb: (be[m], 0, 0)),
            pl.BlockSpec((1, 1, D), lambda m, be, nb: (be[m], 0, 0)),
        ],
        out_specs=pl.BlockSpec((BM, D), lambda m, be, nb: (m, 0)),
    )
    return pl.pallas_call(
        _gmm_body,
        grid_spec=grid_spec,
        out_shape=jax.ShapeDtypeStruct((NP, D), jnp.float32),
    )(bexp, nblk, x_sorted, W1b, b1, W2b, b2)


def kernel(hidden_states, Wg, bg, W1, b1, W2, b2):
    x = hidden_states
    i1, i2, w0, w1, re, ro, cnt = _router(x, Wg, bg)

    counts = cnt.reshape(E)
    blocks_per_e = (counts + BM - 1) // BM
    block_end = jnp.cumsum(blocks_per_e)
    row_start = (block_end - blocks_per_e) * BM
    nblk = block_end[-1:].astype(jnp.int32)
    m_idx = jnp.arange(NB, dtype=jnp.int32)
    bexp = jnp.minimum(
        jnp.sum((m_idx[:, None] >= block_end[None, :]).astype(jnp.int32), 1),
        E - 1,
    )

    eye = jnp.arange(E, dtype=jnp.int32)[None, :]
    rs1 = jnp.sum(jnp.where(i1 == eye, row_start[None, :], 0), axis=1)
    rs2 = jnp.sum(jnp.where(i2 == eye, row_start[None, :], 0), axis=1)
    de = (rs1 + re[:, 0]).reshape(NW, TPW)
    do = (rs2 + ro[:, 0]).reshape(NW, TPW)

    x_sorted = _dispatch(x, de, do)

    y = _gmm(x_sorted, W1.astype(jnp.bfloat16), b1.reshape(E, 1, 2 * I),
             W2.astype(jnp.bfloat16), b2.reshape(E, 1, D), bexp, nblk)

    return _cgather(y, de, do, w0.reshape(NW, TPW, 16),
                    w1.reshape(NW, TPW, 16))
